# baseline (speedup 1.0000x reference)
import jax
import jax.numpy as jnp
from jax import lax
from jax.experimental import pallas as pl
from jax.experimental.pallas import tpu as pltpu
from jax.experimental.pallas import tpu_sc as plsc

B = 16384
D = 64
NC = 2
NS = 16
NW = NC * NS
CHUNK = 128
ROWS_PER_W = B // NW
NCHUNK = ROWS_PER_W // CHUNK
NBUF = 3
LANES = 16


def _swap_halves(cbuf, gbuf):
    def row(i, carry):
        for c in range(D // LANES):
            hi = pl.ds(D + c * LANES, LANES)
            lo = pl.ds(c * LANES, LANES)
            t = cbuf[i, hi]
            cbuf[i, hi] = gbuf[i, lo]
            gbuf[i, lo] = t
        return carry
    lax.fori_loop(0, CHUNK, row, 0)


def _gather_body(ct_hbm, g_hbm, wct, wg,
                 out0, out1,
                 cti_v, gi_v, cbuf, gbuf, gsem, ssem):
    wid = lax.axis_index("s") * NC + lax.axis_index("c")
    grp = (wid // 8) * 8
    row = wid % 8
    base = wid * ROWS_PER_W

    pltpu.sync_copy(ct_hbm.at[pl.ds(grp, 8)], cti_v)
    pltpu.sync_copy(g_hbm.at[pl.ds(grp, 8)], gi_v)

    def fire_gathers(j):
        s = j % NBUF
        cols = pl.ds(j * CHUNK, CHUNK)
        return (pltpu.async_copy(wct.at[cti_v.at[row, cols]], cbuf.at[s], gsem.at[s]),
                pltpu.async_copy(wg.at[gi_v.at[row, cols]], gbuf.at[s], gsem.at[s]))

    gth = [None] * NCHUNK
    sto = [None] * NCHUNK
    for j in range(NBUF):
        gth[j] = fire_gathers(j)
    for j in range(NCHUNK):
        s = j % NBUF
        gth[j][0].wait()
        gth[j][1].wait()
        _swap_halves(cbuf.at[s], gbuf.at[s])
        rows = pl.ds(base + j * CHUNK, CHUNK)
        sto[j] = (pltpu.async_copy(cbuf.at[s], out0.at[rows], ssem.at[s]),
                  pltpu.async_copy(gbuf.at[s], out1.at[rows], ssem.at[s]))
        if j + NBUF < NCHUNK:
            sto[j][0].wait()
            sto[j][1].wait()
            gth[j + NBUF] = fire_gathers(j + NBUF)
    for j in range(max(0, NCHUNK - NBUF), NCHUNK):
        sto[j][0].wait()
        sto[j][1].wait()


def kernel(celltype, gene, W_celltype_0, W_celltype_1, W_gene_0, W_gene_1):
    ct2 = celltype.astype(jnp.int32).reshape(NW, ROWS_PER_W)
    g2 = gene.astype(jnp.int32).reshape(NW, ROWS_PER_W)
    wct = jnp.concatenate([W_celltype_0, W_celltype_1], axis=1)
    wg = jnp.concatenate([W_gene_0, W_gene_1], axis=1)

    mesh = plsc.VectorSubcoreMesh(core_axis_name="c", subcore_axis_name="s")
    out0, out1 = pl.kernel(
        _gather_body,
        out_type=(
            jax.ShapeDtypeStruct((B, 2 * D), jnp.float32),
            jax.ShapeDtypeStruct((B, 2 * D), jnp.float32),
        ),
        mesh=mesh,
        scratch_types=[
            pltpu.VMEM((8, ROWS_PER_W), jnp.int32),
            pltpu.VMEM((8, ROWS_PER_W), jnp.int32),
            pltpu.VMEM((NBUF, CHUNK, 2 * D), jnp.float32),
            pltpu.VMEM((NBUF, CHUNK, 2 * D), jnp.float32),
            pltpu.SemaphoreType.DMA((NBUF,)),
            pltpu.SemaphoreType.DMA((NBUF,)),
        ],
        compiler_params=pltpu.CompilerParams(use_tc_tiling_on_sc=True),
        name="embedding_dict_sc",
    )(ct2, g2, wct, wg)

    return (out0.reshape(B, 2, D), out1.reshape(B, 2, D))
